# speedup vs baseline: 1.3730x; 1.3730x over previous
import jax
import jax.numpy as jnp
from jax.experimental import pallas as pl

_N = 10000
_E = 320000
_G = 64


def _mm_body(x_ref, w_ref, o_ref):
    o_ref[...] = jnp.dot(x_ref[...], w_ref[...],
                         preferred_element_type=jnp.float32)


def _matmul(x, w):
    m, k = x.shape
    n = w.shape[1]
    bm = 2000
    return pl.pallas_call(
        _mm_body,
        grid=(m // bm,),
        in_specs=[pl.BlockSpec((bm, k), lambda i: (i, 0)),
                  pl.BlockSpec((k, n), lambda i: (0, 0))],
        out_specs=pl.BlockSpec((bm, n), lambda i: (i, 0)),
        out_shape=jax.ShapeDtypeStruct((m, n), jnp.float32),
    )(x, w)


def kernel(x, edge_index, edge_weights, batch, W1, b1, g1, be1, W2, b2, g2,
           be2, W3, b3, fc1_W, fc1_b, fc2_W, fc2_b):
    src = edge_index[0]
    dst = edge_index[1]
    deg = jax.ops.segment_sum(edge_weights, dst, num_segments=_N) + 1.0
    dinv = jax.lax.rsqrt(deg)
    norm = dinv[src] * edge_weights * dinv[dst]
    selfw = dinv * dinv

    def conv(h, W, b):
        hw = _matmul(h, W)
        msg = jnp.take(hw, src, axis=0) * norm[:, None]
        agg = jax.ops.segment_sum(msg, dst, num_segments=_N)
        return agg + hw * selfw[:, None] + b

    def bn(h, g, beta):
        mean = jnp.mean(h, axis=0)
        var = jnp.var(h, axis=0)
        return (h - mean) * jax.lax.rsqrt(var + 1e-5) * g + beta

    h = bn(conv(x, W1, b1), g1, be1)
    h = bn(conv(h, W2, b2), g2, be2)
    h = jax.nn.relu(conv(h, W3, b3))
    pooled = jax.ops.segment_max(h, batch, num_segments=_G)
    pooled = jnp.where(jnp.isfinite(pooled), pooled, 0.0)
    z = jax.nn.relu(pooled @ fc1_W + fc1_b)
    out = z @ fc2_W + fc2_b
    return out.reshape(-1)


# speedup vs baseline: 7.2086x; 5.2503x over previous
import functools

import jax
import jax.numpy as jnp
from jax import lax
from jax.experimental import pallas as pl
from jax.experimental.pallas import tpu as pltpu
from jax.experimental.pallas import tpu_sc as plsc

_N = 10000
_NP = 10240
_E = 320000
_EP = 331776
_G = 64
_NW = 32
_EPW = _EP // _NW
_CH = 128
_NCH = _EPW // _CH
_BM = 1024
_NBLK = _NP // _BM

_sc_mesh = plsc.VectorSubcoreMesh(core_axis_name="c", subcore_axis_name="s")
_sc_params = pltpu.CompilerParams(needs_layout_passes=False)


def _wid():
    return lax.axis_index("s") * 2 + lax.axis_index("c")


@functools.partial(
    pl.kernel,
    out_type=jax.ShapeDtypeStruct((_NW, 80, 128), jnp.float32),
    mesh=_sc_mesh,
    compiler_params=_sc_params,
    scratch_types=[
        pltpu.VMEM((80, 128), jnp.float32),
        pltpu.VMEM((_CH,), jnp.int32),
        pltpu.VMEM((_CH,), jnp.float32),
    ],
)
def _deg_kernel(dst_hbm, ew_hbm, degp_hbm, acc, dstb, ewb):
    w = _wid()

    def zbody(i, _):
        acc[i // 8, pl.ds((i % 8) * 16, 16)] = jnp.zeros((16,), jnp.float32)
        return 0

    lax.fori_loop(0, 80 * 8, zbody, 0)

    def cbody(c, _):
        b = w * _EPW + c * _CH
        pltpu.sync_copy(dst_hbm.at[pl.ds(b, _CH)], dstb)
        pltpu.sync_copy(ew_hbm.at[pl.ds(b, _CH)], ewb)

        def ibody(i, _):
            idx = dstb[pl.ds(i * 16, 16)]
            val = ewb[pl.ds(i * 16, 16)]
            plsc.addupdate_scatter(
                acc, [jnp.right_shift(idx, 7), jnp.bitwise_and(idx, 127)],
                val)
            return 0

        lax.fori_loop(0, _CH // 16, ibody, 0)
        return 0

    lax.fori_loop(0, _NCH, cbody, 0)
    pltpu.sync_copy(acc, degp_hbm.at[w])


def _dinv_body(degp_ref, dinv_ref):
    deg = jnp.sum(degp_ref[...], axis=0)
    dinv_ref[...] = jnp.where(deg > 0, lax.rsqrt(deg), 0.0)


def _dinv_call(degp):
    return pl.pallas_call(
        _dinv_body,
        out_shape=jax.ShapeDtypeStruct((80, 128), jnp.float32),
    )(degp)


@functools.partial(
    pl.kernel,
    out_type=jax.ShapeDtypeStruct((_EP,), jnp.float32),
    mesh=_sc_mesh,
    compiler_params=_sc_params,
    scratch_types=[
        pltpu.VMEM((80, 128), jnp.float32),
        pltpu.VMEM((_CH,), jnp.int32),
        pltpu.VMEM((_CH,), jnp.int32),
        pltpu.VMEM((_CH,), jnp.float32),
        pltpu.VMEM((_CH,), jnp.float32),
    ],
)
def _norm_kernel(src_hbm, dst_hbm, ew_hbm, dinv_hbm, norm_hbm,
                 dinv_v, srcb, dstb, ewb, normb):
    w = _wid()
    pltpu.sync_copy(dinv_hbm, dinv_v)

    def cbody(c, _):
        b = w * _EPW + c * _CH
        pltpu.sync_copy(src_hbm.at[pl.ds(b, _CH)], srcb)
        pltpu.sync_copy(dst_hbm.at[pl.ds(b, _CH)], dstb)
        pltpu.sync_copy(ew_hbm.at[pl.ds(b, _CH)], ewb)

        def ibody(i, _):
            s = srcb[pl.ds(i * 16, 16)]
            d = dstb[pl.ds(i * 16, 16)]
            e = ewb[pl.ds(i * 16, 16)]
            ds_ = plsc.load_gather(
                dinv_v, [jnp.right_shift(s, 7), jnp.bitwise_and(s, 127)])
            dd = plsc.load_gather(
                dinv_v, [jnp.right_shift(d, 7), jnp.bitwise_and(d, 127)])
            normb[pl.ds(i * 16, 16)] = ds_ * e * dd
            return 0

        lax.fori_loop(0, _CH // 16, ibody, 0)
        pltpu.sync_copy(normb, norm_hbm.at[pl.ds(b, _CH)])
        return 0

    lax.fori_loop(0, _NCH, cbody, 0)


_ROWS_PER_TILE = _NP // 16
_ZR = 64


@functools.partial(
    pl.kernel,
    out_type=jax.ShapeDtypeStruct((2, _NP, 128), jnp.float32),
    mesh=_sc_mesh,
    compiler_params=_sc_params,
    scratch_types=[
        pltpu.VMEM_SHARED((_NP, 128), jnp.float32),
        pltpu.VMEM((_CH,), jnp.int32),
        pltpu.VMEM((_CH,), jnp.int32),
        pltpu.VMEM((_CH,), jnp.float32),
        pltpu.VMEM((_CH, 128), jnp.float32),
        pltpu.VMEM((_ZR, 128), jnp.float32),
        pltpu.SemaphoreType.DMA,
    ],
)
def _agg_kernel(hw_hbm, src_hbm, dst_hbm, norm_hbm, outp_hbm,
                acc_sh, srcb, dstb, normb, rows, zbuf, sem):
    cid = lax.axis_index("c")
    sid = lax.axis_index("s")
    w = sid * 2 + cid

    def zb(i, _):
        zbuf[i // 8, pl.ds((i % 8) * 16, 16)] = jnp.zeros((16,), jnp.float32)
        return 0

    lax.fori_loop(0, _ZR * 8, zb, 0)

    def zc(k, _):
        pltpu.sync_copy(zbuf, acc_sh.at[pl.ds(sid * _ROWS_PER_TILE
                                              + k * _ZR, _ZR)])
        return 0

    lax.fori_loop(0, _ROWS_PER_TILE // _ZR, zc, 0)
    plsc.subcore_barrier()

    def cbody(c, _):
        b = w * _EPW + c * _CH
        pltpu.sync_copy(src_hbm.at[pl.ds(b, _CH)], srcb)
        pltpu.sync_copy(dst_hbm.at[pl.ds(b, _CH)], dstb)
        pltpu.sync_copy(norm_hbm.at[pl.ds(b, _CH)], normb)
        pltpu.async_copy(hw_hbm.at[srcb], rows, sem).wait()

        def rbody(r, _):
            ridx = jnp.zeros((16,), jnp.int32) + r
            nv = plsc.load_gather(normb, [ridx])
            for c8 in range(8):
                sl = rows[r, pl.ds(c8 * 16, 16)]
                rows[r, pl.ds(c8 * 16, 16)] = sl * nv
            return 0

        lax.fori_loop(0, _CH, rbody, 0)
        pltpu.sync_copy(rows, acc_sh.at[dstb], add=True)
        return 0

    lax.fori_loop(0, _NCH, cbody, 0)
    plsc.subcore_barrier()
    pltpu.sync_copy(
        acc_sh.at[pl.ds(sid * _ROWS_PER_TILE, _ROWS_PER_TILE)],
        outp_hbm.at[cid, pl.ds(sid * _ROWS_PER_TILE, _ROWS_PER_TILE)])


def _mm_body(x_ref, w_ref, o_ref):
    o_ref[...] = jnp.dot(x_ref[...], w_ref[...],
                         preferred_element_type=jnp.float32)


def _mm(x, w):
    n = w.shape[1]
    return pl.pallas_call(
        _mm_body,
        grid=(_NBLK,),
        in_specs=[pl.BlockSpec((_BM, 128), lambda i: (i, 0)),
                  pl.BlockSpec((128, n), lambda i: (0, 0))],
        out_specs=pl.BlockSpec((_BM, n), lambda i: (i, 0)),
        out_shape=jax.ShapeDtypeStruct((_NP, n), jnp.float32),
    )(x, w)


def _mmbn_body(h_ref, st_ref, g_ref, be_ref, w_ref, o_ref):
    s = st_ref[0:1, :]
    ss = st_ref[1:2, :]
    mean = s * (1.0 / _N)
    var = ss * (1.0 / _N) - mean * mean
    scale = lax.rsqrt(var + 1e-5) * g_ref[...]
    shift = be_ref[...] - mean * scale
    hn = h_ref[...] * scale + shift
    o_ref[...] = jnp.dot(hn, w_ref[...], preferred_element_type=jnp.float32)


def _mmbn(h, st, g, be, w):
    n = w.shape[1]
    return pl.pallas_call(
        _mmbn_body,
        grid=(_NBLK,),
        in_specs=[pl.BlockSpec((_BM, 128), lambda i: (i, 0)),
                  pl.BlockSpec((8, 128), lambda i: (0, 0)),
                  pl.BlockSpec((128,), lambda i: (0,)),
                  pl.BlockSpec((128,), lambda i: (0,)),
                  pl.BlockSpec((128, n), lambda i: (0, 0))],
        out_specs=pl.BlockSpec((_BM, n), lambda i: (i, 0)),
        out_shape=jax.ShapeDtypeStruct((_NP, n), jnp.float32),
    )(h, st, g, be, w)


def _comb_body(outp_ref, b_ref, o_ref, st_ref, acc_ref):
    i = pl.program_id(0)
    v = outp_ref[0] + outp_ref[1] + b_ref[...]
    row = i * _BM + lax.broadcasted_iota(jnp.int32, (_BM, 1), 0)
    v = jnp.where(row < _N, v, 0.0)
    o_ref[...] = v

    @pl.when(i == 0)
    def _():
        acc_ref[...] = jnp.zeros_like(acc_ref)

    acc_ref[0:1, :] += jnp.sum(v, axis=0, keepdims=True)
    acc_ref[1:2, :] += jnp.sum(v * v, axis=0, keepdims=True)

    @pl.when(i == _NBLK - 1)
    def _():
        st_ref[...] = acc_ref[...]


def _comb(outp, b):
    return pl.pallas_call(
        _comb_body,
        grid=(_NBLK,),
        in_specs=[pl.BlockSpec((2, _BM, 128), lambda i: (0, i, 0)),
                  pl.BlockSpec((128,), lambda i: (0,))],
        out_specs=(pl.BlockSpec((_BM, 128), lambda i: (i, 0)),
                   pl.BlockSpec((8, 128), lambda i: (0, 0))),
        out_shape=(jax.ShapeDtypeStruct((_NP, 128), jnp.float32),
                   jax.ShapeDtypeStruct((8, 128), jnp.float32)),
        scratch_shapes=[pltpu.VMEM((8, 128), jnp.float32)],
    )(outp, b)


def _pool_body(outp_ref, b_ref, batch_ref, f1w_ref, f1b_ref, f2w_ref,
               f2b_ref, o_ref, pacc_ref):
    i = pl.program_id(0)
    v = outp_ref[0] + outp_ref[1] + b_ref[...]
    v = jnp.maximum(v, 0.0)
    row = i * _BM + lax.broadcasted_iota(jnp.int32, (_BM, 1), 0)
    v = jnp.where(row < _N, v, -jnp.inf)
    bt = batch_ref[...]

    @pl.when(i == 0)
    def _():
        pacc_ref[...] = jnp.full_like(pacc_ref, -jnp.inf)

    giota = lax.broadcasted_iota(jnp.int32, (_G, 1), 0)

    def gbody(g, _):
        m = bt == g
        mx = jnp.max(jnp.where(m, v, -jnp.inf), axis=0, keepdims=True)
        pacc_ref[...] = jnp.where(giota == g,
                                  jnp.maximum(pacc_ref[...], mx),
                                  pacc_ref[...])
        return 0

    lax.fori_loop(0, _G, gbody, 0)

    @pl.when(i == _NBLK - 1)
    def _():
        pooled = pacc_ref[...][:, :64]
        pooled = jnp.where(jnp.isfinite(pooled), pooled, 0.0)
        z = jnp.maximum(
            jnp.dot(pooled, f1w_ref[...],
                    preferred_element_type=jnp.float32) + f1b_ref[...], 0.0)
        o_ref[...] = jnp.dot(z, f2w_ref[...],
                             preferred_element_type=jnp.float32) + f2b_ref[...]


def _pool(outp, b, batch2, f1w, f1b, f2w, f2b):
    return pl.pallas_call(
        _pool_body,
        grid=(_NBLK,),
        in_specs=[pl.BlockSpec((2, _BM, 128), lambda i: (0, i, 0)),
                  pl.BlockSpec((128,), lambda i: (0,)),
                  pl.BlockSpec((_BM, 1), lambda i: (i, 0)),
                  pl.BlockSpec((64, 32), lambda i: (0, 0)),
                  pl.BlockSpec((32,), lambda i: (0,)),
                  pl.BlockSpec((32, 1), lambda i: (0, 0)),
                  pl.BlockSpec((1,), lambda i: (0,))],
        out_specs=pl.BlockSpec((_G, 1), lambda i: (0, 0)),
        out_shape=jax.ShapeDtypeStruct((_G, 1), jnp.float32),
        scratch_shapes=[pltpu.VMEM((_G, 128), jnp.float32)],
    )(outp, b, batch2, f1w, f1b, f2w, f2b)


def kernel(x, edge_index, edge_weights, batch, W1, b1, g1, be1, W2, b2, g2,
           be2, W3, b3, fc1_W, fc1_b, fc2_W, fc2_b):
    loop = jnp.arange(_NP, dtype=jnp.int32)
    epad = _EP - _E - _NP
    src = jnp.concatenate([edge_index[0], loop,
                           jnp.zeros((epad,), jnp.int32)])
    dst = jnp.concatenate([edge_index[1], loop,
                           jnp.zeros((epad,), jnp.int32)])
    ew = jnp.concatenate([edge_weights, jnp.ones((_NP,), jnp.float32),
                          jnp.zeros((epad,), jnp.float32)])
    xp = jnp.concatenate([x, jnp.zeros((_NP - _N, 128), jnp.float32)])
    batch2 = jnp.concatenate(
        [batch, jnp.full((_NP - _N,), _G - 1, jnp.int32)])[:, None]

    degp = _deg_kernel(dst, ew)
    dinv = _dinv_call(degp)
    norm = _norm_kernel(src, dst, ew, dinv)

    hw1 = _mm(xp, W1)
    p1 = _agg_kernel(hw1, src, dst, norm)
    h1, st1 = _comb(p1, b1)

    hw2 = _mmbn(h1, st1, g1, be1, W2)
    p2 = _agg_kernel(hw2, src, dst, norm)
    h2, st2 = _comb(p2, b2)

    W3p = jnp.concatenate([W3, jnp.zeros((128, 64), jnp.float32)], axis=1)
    b3p = jnp.concatenate([b3, jnp.zeros((64,), jnp.float32)])
    hw3 = _mmbn(h2, st2, g2, be2, W3p)
    p3 = _agg_kernel(hw3, src, dst, norm)
    out = _pool(p3, b3p, batch2, fc1_W, fc1_b, fc2_W, fc2_b)
    return out.reshape(-1)
